# baseline (speedup 1.0000x reference)
import jax
import jax.numpy as jnp
from jax import lax
from jax.experimental import pallas as pl
from jax.experimental.pallas import tpu as pltpu

_DIM = 512
_NUM_ITEM = 4096
_K = 16
_TR = 256


def _tile_body(x_ref, mp_ref, mpb_ref, out1_ref, out2_ref, att_s):
    j = pl.program_id(0)

    qc = x_ref[0]
    attn = lax.dot_general(
        qc,
        mp_ref[...],
        (((0,), (1,)), ((), ())),
        preferred_element_type=jnp.float32,
    )
    att_s[pl.ds(j % 2, 1)] = attn[None]

    att = att_s[(j - 1) % 2]
    m0 = jnp.max(att, axis=1, keepdims=True)
    denom = jnp.ones((_TR, 1), jnp.float32)
    m = m0
    for _ in range(_K - 1):
        m = jnp.max(
            jnp.where(att < m, att, -jnp.inf), axis=1, keepdims=True
        )
        denom = denom + jnp.exp(m - m0)
    t = m
    p = jnp.where(att >= t, jnp.exp(att - m0), 0.0)
    recip = 1.0 / denom
    out2_ref[...] = p * att * recip
    out1t = lax.dot_general(
        mpb_ref[...],
        p.astype(jnp.bfloat16),
        (((0,), (1,)), ((), ())),
        preferred_element_type=jnp.float32,
    )
    out1_ref[0] = out1t * jnp.reshape(recip, (1, _TR))


def kernel(input, mempool):
    B, C, H, W = input.shape
    x3 = input.reshape(B, C, H * W)
    rows = B * H * W
    ntiles = rows // _TR
    tpb = (H * W) // _TR

    def prod_map(i):
        ic = jnp.minimum(i, ntiles - 1)
        return (ic // tpb, 0, ic % tpb)

    def cons_map3(i):
        ic = jnp.maximum(i - 1, 0)
        return (ic // tpb, 0, ic % tpb)

    def cons_map2(i):
        return (jnp.maximum(i - 1, 0), 0)

    out1, out2 = pl.pallas_call(
        _tile_body,
        grid=(ntiles + 1,),
        in_specs=[
            pl.BlockSpec((1, C, _TR), prod_map),
            pl.BlockSpec((_NUM_ITEM, C), lambda i: (0, 0)),
            pl.BlockSpec((_NUM_ITEM, C), lambda i: (0, 0)),
        ],
        out_specs=[
            pl.BlockSpec((1, C, _TR), cons_map3),
            pl.BlockSpec((_TR, _NUM_ITEM), cons_map2),
        ],
        out_shape=[
            jax.ShapeDtypeStruct((B, C, H * W), jnp.float32),
            jax.ShapeDtypeStruct((rows, _NUM_ITEM), jnp.float32),
        ],
        scratch_shapes=[
            pltpu.VMEM((2, _TR, _NUM_ITEM), jnp.float32),
        ],
        compiler_params=pltpu.CompilerParams(
            dimension_semantics=("arbitrary",),
        ),
    )(x3, mempool, mempool.astype(jnp.bfloat16))
    return out1.reshape(B, C, H, W), out2


# speedup vs baseline: 1.0645x; 1.0645x over previous
import jax
import jax.numpy as jnp
from jax import lax
from jax.experimental import pallas as pl
from jax.experimental.pallas import tpu as pltpu

_DIM = 512
_NUM_ITEM = 4096
_K = 16
_TR = 512


def _tile_body(x_ref, mp_ref, mpb_ref, out1_ref, out2_ref, att_s):
    qc = x_ref[0]
    att = lax.dot_general(
        qc,
        mp_ref[...],
        (((0,), (1,)), ((), ())),
        preferred_element_type=jnp.float32,
    )
    att_s[...] = att
    att = att_s[...]
    m0 = jnp.max(att, axis=1, keepdims=True)
    denom = jnp.ones((_TR, 1), jnp.float32)
    m = m0
    for _ in range(_K - 1):
        m = jnp.max(
            jnp.where(att < m, att, -jnp.inf), axis=1, keepdims=True
        )
        denom = denom + jnp.exp(m - m0)
    t = m
    p = jnp.where(att >= t, jnp.exp(att - m0), 0.0)
    recip = 1.0 / denom
    out2_ref[...] = p * att * recip
    out1t = lax.dot_general(
        mpb_ref[...],
        p.astype(jnp.bfloat16),
        (((0,), (1,)), ((), ())),
        preferred_element_type=jnp.float32,
    )
    out1_ref[0] = out1t * jnp.reshape(recip, (1, _TR))


def kernel(input, mempool):
    B, C, H, W = input.shape
    x3 = input.reshape(B, C, H * W)
    rows = B * H * W
    ntiles = rows // _TR
    tpb = (H * W) // _TR
    out1, out2 = pl.pallas_call(
        _tile_body,
        grid=(ntiles,),
        in_specs=[
            pl.BlockSpec((1, C, _TR), lambda i: (i // tpb, 0, i % tpb)),
            pl.BlockSpec((_NUM_ITEM, C), lambda i: (0, 0)),
            pl.BlockSpec((_NUM_ITEM, C), lambda i: (0, 0)),
        ],
        out_specs=[
            pl.BlockSpec((1, C, _TR), lambda i: (i // tpb, 0, i % tpb)),
            pl.BlockSpec((_TR, _NUM_ITEM), lambda i: (i, 0)),
        ],
        out_shape=[
            jax.ShapeDtypeStruct((B, C, H * W), jnp.float32),
            jax.ShapeDtypeStruct((rows, _NUM_ITEM), jnp.float32),
        ],
        scratch_shapes=[
            pltpu.VMEM((_TR, _NUM_ITEM), jnp.float32),
        ],
        compiler_params=pltpu.CompilerParams(
            dimension_semantics=("arbitrary",),
        ),
    )(x3, mempool, mempool.astype(jnp.bfloat16))
    return out1.reshape(B, C, H, W), out2


# speedup vs baseline: 1.6614x; 1.5608x over previous
import jax
import jax.numpy as jnp
from jax import lax
from jax.experimental import pallas as pl
from jax.experimental.pallas import tpu as pltpu

_DIM = 512
_NUM_ITEM = 4096
_K = 16
_TR = 512


def _tile_body(x_ref, mp_ref, mpb_ref, out1_ref, out2_ref, att_s):
    qc = x_ref[0]
    att = lax.dot_general(
        qc,
        mp_ref[...],
        (((0,), (1,)), ((), ())),
        preferred_element_type=jnp.float32,
    )
    att_s[...] = att
    att = att_s[...]
    nsl = _NUM_ITEM // 128
    sl = [att[:, 128 * g : 128 * (g + 1)] for g in range(nsl)]

    def _treemax(vals):
        while len(vals) > 1:
            vals = [
                jnp.maximum(vals[i], vals[i + 1])
                for i in range(0, len(vals) - 1, 2)
            ] + ([vals[-1]] if len(vals) % 2 else [])
        return vals[0]

    g1 = _treemax(sl)
    levels = [g1]
    prev = g1
    for _lvl in range(3):
        cand = [jnp.where(s < prev, s, -jnp.inf) for s in sl]
        prev = _treemax(cand)
        levels.append(prev)
    gcat = jnp.concatenate(levels, axis=1)
    m0 = jnp.max(gcat, axis=1, keepdims=True)
    m = m0
    for _ in range(_K - 1):
        m = jnp.max(
            jnp.where(gcat < m, gcat, -jnp.inf), axis=1, keepdims=True
        )
    t = m
    p = jnp.where(att >= t, jnp.exp(att - m0), 0.0)
    denom = jnp.sum(p, axis=1, keepdims=True)
    recip = 1.0 / denom
    out2_ref[...] = p * att * recip
    out1t = lax.dot_general(
        mpb_ref[...],
        p.astype(jnp.bfloat16),
        (((0,), (1,)), ((), ())),
        preferred_element_type=jnp.float32,
    )
    out1_ref[0] = out1t * jnp.reshape(recip, (1, _TR))


def kernel(input, mempool):
    B, C, H, W = input.shape
    x3 = input.reshape(B, C, H * W)
    rows = B * H * W
    ntiles = rows // _TR
    tpb = (H * W) // _TR
    out1, out2 = pl.pallas_call(
        _tile_body,
        grid=(ntiles,),
        in_specs=[
            pl.BlockSpec((1, C, _TR), lambda i: (i // tpb, 0, i % tpb)),
            pl.BlockSpec((_NUM_ITEM, C), lambda i: (0, 0)),
            pl.BlockSpec((_NUM_ITEM, C), lambda i: (0, 0)),
        ],
        out_specs=[
            pl.BlockSpec((1, C, _TR), lambda i: (i // tpb, 0, i % tpb)),
            pl.BlockSpec((_TR, _NUM_ITEM), lambda i: (i, 0)),
        ],
        out_shape=[
            jax.ShapeDtypeStruct((B, C, H * W), jnp.float32),
            jax.ShapeDtypeStruct((rows, _NUM_ITEM), jnp.float32),
        ],
        scratch_shapes=[
            pltpu.VMEM((_TR, _NUM_ITEM), jnp.float32),
        ],
        compiler_params=pltpu.CompilerParams(
            dimension_semantics=("arbitrary",),
        ),
    )(x3, mempool, mempool.astype(jnp.bfloat16))
    return out1.reshape(B, C, H, W), out2


# speedup vs baseline: 1.7924x; 1.0788x over previous
import jax
import jax.numpy as jnp
from jax import lax
from jax.experimental import pallas as pl
from jax.experimental.pallas import tpu as pltpu

_DIM = 512
_NUM_ITEM = 4096
_K = 16
_TR = 512


def _tile_body(x_ref, mp_ref, mpb_ref, out1_ref, out2_ref, att_s):
    qc = x_ref[0]
    att = lax.dot_general(
        qc,
        mp_ref[...],
        (((0,), (1,)), ((), ())),
        preferred_element_type=jnp.float32,
    )
    att_s[...] = att
    att = att_s[...]
    nsl = _NUM_ITEM // 128
    sl = [att[:, 128 * g : 128 * (g + 1)] for g in range(nsl)]
    a = sl[0]
    neg = jnp.full_like(a, -jnp.inf)
    b, c, d = neg, neg, neg
    for s in sl[1:]:
        na = jnp.maximum(a, s)
        t1 = jnp.minimum(a, s)
        nb = jnp.maximum(b, t1)
        t2 = jnp.minimum(b, t1)
        nc = jnp.maximum(c, t2)
        t3 = jnp.minimum(c, t2)
        d = jnp.maximum(d, t3)
        a, b, c = na, nb, nc
    gcat = jnp.concatenate([a, b, c, d], axis=1)
    m0 = jnp.max(gcat, axis=1, keepdims=True)
    m = m0
    for _ in range(_K - 1):
        m = jnp.max(
            jnp.where(gcat < m, gcat, -jnp.inf), axis=1, keepdims=True
        )
    t = m
    p = jnp.where(att >= t, jnp.exp(att - m0), 0.0)
    denom = jnp.sum(p, axis=1, keepdims=True)
    recip = 1.0 / denom
    out2_ref[...] = p * att * recip
    out1t = lax.dot_general(
        mpb_ref[...],
        p.astype(jnp.bfloat16),
        (((0,), (1,)), ((), ())),
        preferred_element_type=jnp.float32,
    )
    out1_ref[0] = out1t * jnp.reshape(recip, (1, _TR))


def kernel(input, mempool):
    B, C, H, W = input.shape
    x3 = input.reshape(B, C, H * W)
    rows = B * H * W
    ntiles = rows // _TR
    tpb = (H * W) // _TR
    out1, out2 = pl.pallas_call(
        _tile_body,
        grid=(ntiles,),
        in_specs=[
            pl.BlockSpec((1, C, _TR), lambda i: (i // tpb, 0, i % tpb)),
            pl.BlockSpec((_NUM_ITEM, C), lambda i: (0, 0)),
            pl.BlockSpec((_NUM_ITEM, C), lambda i: (0, 0)),
        ],
        out_specs=[
            pl.BlockSpec((1, C, _TR), lambda i: (i // tpb, 0, i % tpb)),
            pl.BlockSpec((_TR, _NUM_ITEM), lambda i: (i, 0)),
        ],
        out_shape=[
            jax.ShapeDtypeStruct((B, C, H * W), jnp.float32),
            jax.ShapeDtypeStruct((rows, _NUM_ITEM), jnp.float32),
        ],
        scratch_shapes=[
            pltpu.VMEM((_TR, _NUM_ITEM), jnp.float32),
        ],
        compiler_params=pltpu.CompilerParams(
            dimension_semantics=("arbitrary",),
        ),
    )(x3, mempool, mempool.astype(jnp.bfloat16))
    return out1.reshape(B, C, H, W), out2


# speedup vs baseline: 1.7941x; 1.0010x over previous
import jax
import jax.numpy as jnp
from jax import lax
from jax.experimental import pallas as pl
from jax.experimental.pallas import tpu as pltpu

_DIM = 512
_NUM_ITEM = 4096
_K = 16
_TR = 512


def _tile_body(x_ref, mp_ref, mpb_ref, out1_ref, out2_ref, att_s):
    qc = x_ref[0]
    att = lax.dot_general(
        qc,
        mp_ref[...],
        (((0,), (1,)), ((), ())),
        preferred_element_type=jnp.float32,
        precision=lax.Precision.DEFAULT,
    )
    att_s[...] = att
    att = att_s[...]
    nsl = _NUM_ITEM // 128
    sl = [att[:, 128 * g : 128 * (g + 1)] for g in range(nsl)]
    a = sl[0]
    neg = jnp.full_like(a, -jnp.inf)
    b, c, d = neg, neg, neg
    for s in sl[1:]:
        na = jnp.maximum(a, s)
        t1 = jnp.minimum(a, s)
        nb = jnp.maximum(b, t1)
        t2 = jnp.minimum(b, t1)
        nc = jnp.maximum(c, t2)
        t3 = jnp.minimum(c, t2)
        d = jnp.maximum(d, t3)
        a, b, c = na, nb, nc
    gcat = jnp.concatenate([a, b, c, d], axis=1)
    m0 = jnp.max(gcat, axis=1, keepdims=True)
    m = m0
    for _ in range(_K - 1):
        m = jnp.max(
            jnp.where(gcat < m, gcat, -jnp.inf), axis=1, keepdims=True
        )
    t = m
    p = jnp.where(att >= t, jnp.exp(att - m0), 0.0)
    denom = jnp.sum(p, axis=1, keepdims=True)
    recip = 1.0 / denom
    out2_ref[...] = p * att * recip
    out1t = lax.dot_general(
        mpb_ref[...],
        p.astype(jnp.bfloat16),
        (((0,), (1,)), ((), ())),
        preferred_element_type=jnp.float32,
    )
    out1_ref[0] = out1t * jnp.reshape(recip, (1, _TR))


def kernel(input, mempool):
    B, C, H, W = input.shape
    x3 = input.reshape(B, C, H * W)
    rows = B * H * W
    ntiles = rows // _TR
    tpb = (H * W) // _TR
    out1, out2 = pl.pallas_call(
        _tile_body,
        grid=(ntiles,),
        in_specs=[
            pl.BlockSpec((1, C, _TR), lambda i: (i // tpb, 0, i % tpb)),
            pl.BlockSpec((_NUM_ITEM, C), lambda i: (0, 0)),
            pl.BlockSpec((_NUM_ITEM, C), lambda i: (0, 0)),
        ],
        out_specs=[
            pl.BlockSpec((1, C, _TR), lambda i: (i // tpb, 0, i % tpb)),
            pl.BlockSpec((_TR, _NUM_ITEM), lambda i: (i, 0)),
        ],
        out_shape=[
            jax.ShapeDtypeStruct((B, C, H * W), jnp.float32),
            jax.ShapeDtypeStruct((rows, _NUM_ITEM), jnp.float32),
        ],
        scratch_shapes=[
            pltpu.VMEM((_TR, _NUM_ITEM), jnp.float32),
        ],
        compiler_params=pltpu.CompilerParams(
            dimension_semantics=("arbitrary",),
        ),
    )(x3, mempool, mempool.astype(jnp.bfloat16))
    return out1.reshape(B, C, H, W), out2


# speedup vs baseline: 1.8105x; 1.0091x over previous
import jax
import jax.numpy as jnp
from jax import lax
from jax.experimental import pallas as pl
from jax.experimental.pallas import tpu as pltpu

_DIM = 512
_NUM_ITEM = 4096
_K = 16
_TR = 512


def _tile_body(x_ref, mp_ref, mpb_ref, out1_ref, out2_ref):
    qc = x_ref[0]
    att = lax.dot_general(
        qc,
        mp_ref[...],
        (((0,), (1,)), ((), ())),
        preferred_element_type=jnp.float32,
    )
    nsl = _NUM_ITEM // 128
    sl = [att[:, 128 * g : 128 * (g + 1)] for g in range(nsl)]
    a = sl[0]
    neg = jnp.full_like(a, -jnp.inf)
    b, c, d = neg, neg, neg
    for s in sl[1:]:
        na = jnp.maximum(a, s)
        t1 = jnp.minimum(a, s)
        nb = jnp.maximum(b, t1)
        t2 = jnp.minimum(b, t1)
        nc = jnp.maximum(c, t2)
        t3 = jnp.minimum(c, t2)
        d = jnp.maximum(d, t3)
        a, b, c = na, nb, nc
    gcat = jnp.concatenate([a, b, c, d], axis=1)
    m0 = jnp.max(gcat, axis=1, keepdims=True)
    m = m0
    for _ in range(_K - 1):
        m = jnp.max(
            jnp.where(gcat < m, gcat, -jnp.inf), axis=1, keepdims=True
        )
    t = m
    p = jnp.where(att >= t, jnp.exp(att - m0), 0.0)
    denom = jnp.sum(p, axis=1, keepdims=True)
    recip = 1.0 / denom
    out2_ref[...] = p * att * recip
    out1t = lax.dot_general(
        mpb_ref[...],
        p.astype(jnp.bfloat16),
        (((0,), (1,)), ((), ())),
        preferred_element_type=jnp.float32,
    )
    out1_ref[0] = out1t * jnp.reshape(recip, (1, _TR))


def kernel(input, mempool):
    B, C, H, W = input.shape
    x3 = input.reshape(B, C, H * W)
    rows = B * H * W
    ntiles = rows // _TR
    tpb = (H * W) // _TR
    out1, out2 = pl.pallas_call(
        _tile_body,
        grid=(ntiles,),
        in_specs=[
            pl.BlockSpec((1, C, _TR), lambda i: (i // tpb, 0, i % tpb)),
            pl.BlockSpec((_NUM_ITEM, C), lambda i: (0, 0)),
            pl.BlockSpec((_NUM_ITEM, C), lambda i: (0, 0)),
        ],
        out_specs=[
            pl.BlockSpec((1, C, _TR), lambda i: (i // tpb, 0, i % tpb)),
            pl.BlockSpec((_TR, _NUM_ITEM), lambda i: (i, 0)),
        ],
        out_shape=[
            jax.ShapeDtypeStruct((B, C, H * W), jnp.float32),
            jax.ShapeDtypeStruct((rows, _NUM_ITEM), jnp.float32),
        ],
        compiler_params=pltpu.CompilerParams(
            dimension_semantics=("arbitrary",),
        ),
    )(x3, mempool, mempool.astype(jnp.bfloat16))
    return out1.reshape(B, C, H, W), out2


# speedup vs baseline: 1.8618x; 1.0283x over previous
import jax
import jax.numpy as jnp
from jax import lax
from jax.experimental import pallas as pl
from jax.experimental.pallas import tpu as pltpu

_DIM = 512
_NUM_ITEM = 4096
_K = 16
_TR = 512


def _tile_body(x_ref, mp_ref, mpb_ref, out1_ref, out2_ref):
    qc = x_ref[0]
    att = lax.dot_general(
        qc,
        mp_ref[...],
        (((0,), (1,)), ((), ())),
        preferred_element_type=jnp.float32,
    )
    nsl = _NUM_ITEM // 128
    sl = [att[:, 128 * g : 128 * (g + 1)] for g in range(nsl)]
    def _cs(x, y):
        return jnp.maximum(x, y), jnp.minimum(x, y)

    def _sort4(w, x, y, z):
        w, x = _cs(w, x)
        y, z = _cs(y, z)
        w, y = _cs(w, y)
        x, z = _cs(x, z)
        x, y = _cs(x, y)
        return w, x, y, z

    def _merge_top4(a, b, cleanup=True):
        c1 = jnp.maximum(a[0], b[3])
        c2 = jnp.maximum(a[1], b[2])
        c3 = jnp.maximum(a[2], b[1])
        c4 = jnp.maximum(a[3], b[0])
        if cleanup:
            c1, c3 = _cs(c1, c3)
            c2, c4 = _cs(c2, c4)
            c1, c2 = _cs(c1, c2)
            c3, c4 = _cs(c3, c4)
        return c1, c2, c3, c4

    groups = [
        _sort4(sl[i], sl[i + 1], sl[i + 2], sl[i + 3])
        for i in range(0, nsl, 4)
    ]
    while len(groups) > 2:
        groups = [
            _merge_top4(groups[i], groups[i + 1])
            for i in range(0, len(groups), 2)
        ]
    top4 = _merge_top4(groups[0], groups[1], cleanup=False)
    gcat = jnp.concatenate(top4, axis=1)
    m0 = jnp.max(gcat, axis=1, keepdims=True)
    m = m0
    for _ in range(_K - 1):
        m = jnp.max(
            jnp.where(gcat < m, gcat, -jnp.inf), axis=1, keepdims=True
        )
    t = m
    p = jnp.where(att >= t, jnp.exp(att - m0), 0.0)
    denom = jnp.sum(p, axis=1, keepdims=True)
    recip = 1.0 / denom
    out2_ref[...] = p * att * recip
    out1t = lax.dot_general(
        mpb_ref[...],
        p.astype(jnp.bfloat16),
        (((0,), (1,)), ((), ())),
        preferred_element_type=jnp.float32,
    )
    out1_ref[0] = out1t * jnp.reshape(recip, (1, _TR))


def kernel(input, mempool):
    B, C, H, W = input.shape
    x3 = input.reshape(B, C, H * W)
    rows = B * H * W
    ntiles = rows // _TR
    tpb = (H * W) // _TR
    out1, out2 = pl.pallas_call(
        _tile_body,
        grid=(ntiles,),
        in_specs=[
            pl.BlockSpec((1, C, _TR), lambda i: (i // tpb, 0, i % tpb)),
            pl.BlockSpec((_NUM_ITEM, C), lambda i: (0, 0)),
            pl.BlockSpec((_NUM_ITEM, C), lambda i: (0, 0)),
        ],
        out_specs=[
            pl.BlockSpec((1, C, _TR), lambda i: (i // tpb, 0, i % tpb)),
            pl.BlockSpec((_TR, _NUM_ITEM), lambda i: (i, 0)),
        ],
        out_shape=[
            jax.ShapeDtypeStruct((B, C, H * W), jnp.float32),
            jax.ShapeDtypeStruct((rows, _NUM_ITEM), jnp.float32),
        ],
        compiler_params=pltpu.CompilerParams(
            dimension_semantics=("arbitrary",),
        ),
    )(x3, mempool, mempool.astype(jnp.bfloat16))
    return out1.reshape(B, C, H, W), out2


# speedup vs baseline: 1.9456x; 1.0450x over previous
import jax
import jax.numpy as jnp
from jax import lax
from jax.experimental import pallas as pl
from jax.experimental.pallas import tpu as pltpu

_DIM = 512
_NUM_ITEM = 4096
_K = 16
_TR = 512


def _tile_body(x_ref, mp_ref, mpb_ref, out1_ref, out2_ref):
    qc = x_ref[0]
    att = lax.dot_general(
        qc,
        mp_ref[...],
        (((0,), (1,)), ((), ())),
        preferred_element_type=jnp.float32,
    )
    nsl = _NUM_ITEM // 128
    sl = [att[:, 128 * g : 128 * (g + 1)] for g in range(nsl)]
    def _cs(x, y):
        return jnp.maximum(x, y), jnp.minimum(x, y)

    def _sort4(w, x, y, z):
        w, x = _cs(w, x)
        y, z = _cs(y, z)
        w, y = _cs(w, y)
        x, z = _cs(x, z)
        x, y = _cs(x, y)
        return w, x, y, z

    def _merge_top4(a, b, cleanup=True):
        c1 = jnp.maximum(a[0], b[3])
        c2 = jnp.maximum(a[1], b[2])
        c3 = jnp.maximum(a[2], b[1])
        c4 = jnp.maximum(a[3], b[0])
        if cleanup:
            c1, c3 = _cs(c1, c3)
            c2, c4 = _cs(c2, c4)
            c1, c2 = _cs(c1, c2)
            c3, c4 = _cs(c3, c4)
        return c1, c2, c3, c4

    groups = [
        _sort4(sl[i], sl[i + 1], sl[i + 2], sl[i + 3])
        for i in range(0, nsl, 4)
    ]
    while len(groups) > 2:
        groups = [
            _merge_top4(groups[i], groups[i + 1])
            for i in range(0, len(groups), 2)
        ]
    top4 = _merge_top4(groups[0], groups[1], cleanup=False)
    gcat = jnp.concatenate(top4, axis=1)
    m0 = jnp.max(gcat, axis=1, keepdims=True)
    m = m0
    denom = jnp.ones((_TR, 1), jnp.float32)
    for _ in range(_K - 1):
        m = jnp.max(
            jnp.where(gcat < m, gcat, -jnp.inf), axis=1, keepdims=True
        )
        denom = denom + jnp.exp(m - m0)
    t = m
    p = jnp.where(att >= t, jnp.exp(att - m0), 0.0)
    recip = 1.0 / denom
    out2_ref[...] = p * att * recip
    out1t = lax.dot_general(
        mpb_ref[...],
        p.astype(jnp.bfloat16),
        (((0,), (1,)), ((), ())),
        preferred_element_type=jnp.float32,
    )
    out1_ref[0] = out1t * jnp.reshape(recip, (1, _TR))


def kernel(input, mempool):
    B, C, H, W = input.shape
    x3 = input.reshape(B, C, H * W)
    rows = B * H * W
    ntiles = rows // _TR
    tpb = (H * W) // _TR
    out1, out2 = pl.pallas_call(
        _tile_body,
        grid=(ntiles,),
        in_specs=[
            pl.BlockSpec((1, C, _TR), lambda i: (i // tpb, 0, i % tpb)),
            pl.BlockSpec((_NUM_ITEM, C), lambda i: (0, 0)),
            pl.BlockSpec((_NUM_ITEM, C), lambda i: (0, 0)),
        ],
        out_specs=[
            pl.BlockSpec((1, C, _TR), lambda i: (i // tpb, 0, i % tpb)),
            pl.BlockSpec((_TR, _NUM_ITEM), lambda i: (i, 0)),
        ],
        out_shape=[
            jax.ShapeDtypeStruct((B, C, H * W), jnp.float32),
            jax.ShapeDtypeStruct((rows, _NUM_ITEM), jnp.float32),
        ],
        compiler_params=pltpu.CompilerParams(
            dimension_semantics=("arbitrary",),
        ),
    )(x3, mempool, mempool.astype(jnp.bfloat16))
    return out1.reshape(B, C, H, W), out2
